# baseline (speedup 1.0000x reference)
import functools

import jax
import jax.numpy as jnp
from jax import lax
from jax.experimental import pallas as pl
from jax.experimental.pallas import tpu as pltpu
from jax.experimental.pallas import tpu_sc as plsc

_TOP_K = 32
_B = 32
_N = 4096
_D = 768
_B_SC = 2
_B_TC = _B - _B_SC
_NTILES = 32
_TPB = _NTILES // _B_SC
_RPT = _N // _TPB
_CH = 64
_LANE = 16
_GT = _B_SC * _TOP_K // _LANE
_GPB = _TOP_K // _LANE


def _topk_kernel(alpha_ref, wts_ref, tki_ref, tkw_ref):
    a = alpha_ref[:]
    b, n = a.shape
    iota = lax.broadcasted_iota(jnp.int32, (b, n), 1)
    kiota = lax.broadcasted_iota(jnp.int32, (b, _TOP_K), 1)
    neg = jnp.float32(-jnp.inf)

    def body(k, carry):
        a, wnum, ind, s, ki, kv = carry
        m = jnp.max(a, axis=1, keepdims=True)
        idx = jnp.min(jnp.where(a == m, iota, n), axis=1, keepdims=True)
        onehot = iota == idx
        wnum = wnum + jnp.where(onehot, m, 0.0)
        ind = ind + jnp.where(onehot, 1.0, 0.0)
        s = s + m
        ksel = kiota == k
        ki = ki + jnp.where(ksel, idx, 0)
        kv = kv + jnp.where(ksel, m, 0.0)
        a = jnp.where(onehot, neg, a)
        return a, wnum, ind, s, ki, kv

    zeros = jnp.zeros_like(a)
    s0 = jnp.zeros((b, 1), jnp.float32)
    ki0 = jnp.zeros((b, _TOP_K), jnp.int32)
    kv0 = jnp.zeros((b, _TOP_K), jnp.float32)
    _, wnum, ind, s, ki, kv = lax.fori_loop(
        0, _TOP_K, body, (a, zeros, zeros, s0, ki0, kv0))
    wts_ref[:, 0, :] = jnp.ones_like(a)
    wts_ref[:, 1, :] = alpha_ref[:]
    wts_ref[:, 2, :] = wnum / (s + 1e-8)
    wts_ref[:, 3, :] = ind
    tki_ref[:] = ki
    tkw_ref[:] = kv / (s + 1e-8)


def _stream_kernel(wts_ref, f_ref, acc_ref):
    i = pl.program_id(0)
    nb = pl.program_id(1)
    w = wts_ref[i].astype(jnp.bfloat16)
    part = jnp.dot(w, f_ref[0].astype(jnp.bfloat16),
                   preferred_element_type=jnp.float32)

    @pl.when(nb == 0)
    def _():
        acc_ref[0] = part

    @pl.when(nb != 0)
    def _():
        acc_ref[0] += part


def _sc_sums_kernel(f_hbm, alpha_hbm, out_hbm, rowbuf, wbuf, accbuf):
    c = lax.axis_index("c")
    s = lax.axis_index("s")
    wid = s * 2 + c
    b = _B_TC + wid // _TPB
    n0 = (wid % _TPB) * _RPT

    pltpu.sync_copy(alpha_hbm.at[b, pl.ds(n0, _RPT)], wbuf)

    zero16 = jnp.zeros((_LANE,), jnp.float32)
    for rrow in range(2):
        for cc in range(_D // _LANE):
            accbuf[rrow, pl.ds(cc * _LANE, _LANE)] = zero16

    npanel = _D // 128

    def chunk_body(ch, carry):
        pltpu.sync_copy(f_hbm.at[b, pl.ds(n0 + ch * _CH, _CH), :], rowbuf)
        for p in range(npanel):
            col0 = p * 128

            def row_body(r, regs, col0=col0, ch=ch):
                wa = plsc.load_gather(
                    wbuf, [jnp.full((_LANE,), ch * _CH + r, jnp.int32)])
                out0 = []
                out1 = []
                for cc in range(8):
                    fv = rowbuf[r, pl.ds(col0 + cc * _LANE, _LANE)]
                    out0.append(regs[0][cc] + fv)
                    out1.append(regs[1][cc] + wa * fv)
                return (tuple(out0), tuple(out1))

            z8 = tuple(zero16 for _ in range(8))
            regs = lax.fori_loop(0, _CH, row_body, (z8, z8))
            for k in range(2):
                for cc in range(8):
                    sl = pl.ds(col0 + cc * _LANE, _LANE)
                    accbuf[k, sl] = accbuf[k, sl] + regs[k][cc]
        return carry

    lax.fori_loop(0, _RPT // _CH, chunk_body, 0)
    pltpu.sync_copy(accbuf, out_hbm.at[wid])


def _sc_topk_kernel(f_hbm, tki_hbm, tkw_hbm, out_hbm, idxbuf, wtbuf, rowbuf,
                    accbuf, sem):
    c = lax.axis_index("c")
    s = lax.axis_index("s")
    wid = s * 2 + c

    @pl.when(wid < _GT)
    def _():
        b = _B_TC + wid // _GPB
        k0 = (wid % _GPB) * _LANE
        pltpu.sync_copy(tki_hbm.at[b, pl.ds(k0, _LANE)], idxbuf)
        pltpu.sync_copy(tkw_hbm.at[b, pl.ds(k0, _LANE)], wtbuf)
        pltpu.async_copy(f_hbm.at[b].at[idxbuf], rowbuf, sem).wait()

        zero16 = jnp.zeros((_LANE,), jnp.float32)
        for rrow in range(2):
            for cc in range(_D // _LANE):
                accbuf[rrow, pl.ds(cc * _LANE, _LANE)] = zero16

        npanel = _D // 128
        for p in range(npanel):
            col0 = p * 128

            def row_body(r, regs, col0=col0):
                ww = plsc.load_gather(
                    wtbuf, [jnp.full((_LANE,), r, jnp.int32)])
                out0 = []
                out1 = []
                for cc in range(8):
                    fv = rowbuf[r, pl.ds(col0 + cc * _LANE, _LANE)]
                    out0.append(regs[0][cc] + ww * fv)
                    out1.append(regs[1][cc] + fv)
                return (tuple(out0), tuple(out1))

            z8 = tuple(jnp.zeros((_LANE,), jnp.float32) for _ in range(8))
            regs = lax.fori_loop(0, _LANE, row_body, (z8, z8))
            for k in range(2):
                for cc in range(8):
                    sl = pl.ds(col0 + cc * _LANE, _LANE)
                    accbuf[k, sl] = accbuf[k, sl] + regs[k][cc]

        pltpu.sync_copy(accbuf, out_hbm.at[wid])


_sc_sums = functools.partial(
    pl.kernel,
    out_type=jax.ShapeDtypeStruct((_NTILES, 2, _D), jnp.float32),
    mesh=plsc.VectorSubcoreMesh(core_axis_name="c", subcore_axis_name="s"),
    scratch_types=[
        pltpu.VMEM((_CH, _D), jnp.float32),
        pltpu.VMEM((_RPT,), jnp.float32),
        pltpu.VMEM((2, _D), jnp.float32),
    ],
    compiler_params=pltpu.CompilerParams(needs_layout_passes=False),
)(_sc_sums_kernel)

_sc_topk = functools.partial(
    pl.kernel,
    out_type=jax.ShapeDtypeStruct((_GT, 2, _D), jnp.float32),
    mesh=plsc.VectorSubcoreMesh(core_axis_name="c", subcore_axis_name="s"),
    scratch_types=[
        pltpu.VMEM((_LANE,), jnp.int32),
        pltpu.VMEM((_LANE,), jnp.float32),
        pltpu.VMEM((_LANE, _D), jnp.float32),
        pltpu.VMEM((2, _D), jnp.float32),
        pltpu.SemaphoreType.DMA,
    ],
    compiler_params=pltpu.CompilerParams(needs_layout_passes=False),
)(_sc_topk_kernel)


def _finish_kernel(acc_ref, sc1_ref, sc2_ref, wts_ref, wv_ref, bv_ref,
                   f1w_ref, f1b_ref, f2w_ref, f2b_ref, out_ref):
    a01 = jnp.sum(sc1_ref[:].reshape(_B_SC, _TPB, 2, _D), axis=1)
    a23 = jnp.sum(sc2_ref[:].reshape(_B_SC, _GPB, 2, _D), axis=1)
    acc_sc = jnp.concatenate([a01, a23], axis=1)
    acc = jnp.concatenate([acc_ref[:], acc_sc], axis=0)
    b = acc.shape[0]
    n = wts_ref.shape[2]
    p = jnp.dot(acc.reshape(b * 4, -1), wv_ref[:],
                preferred_element_type=jnp.float32).reshape(b, 4, -1)
    bv = bv_ref[:]
    sum_alpha = jnp.sum(wts_ref[:, 1, :], axis=1, keepdims=True)
    sum_w = jnp.sum(wts_ref[:, 2, :], axis=1, keepdims=True)
    s_all = p[:, 0, :] + jnp.float32(n) * bv
    broad = p[:, 1, :] + sum_alpha * bv
    focus = p[:, 2, :] + sum_w * bv
    s_top = p[:, 3, :] + jnp.float32(_TOP_K) * bv
    periph = (s_all - s_top) * jnp.float32(1.0 / (n - _TOP_K))
    concat = jnp.concatenate([focus, broad, periph], axis=-1)
    h = jnp.dot(concat, f1w_ref[:], preferred_element_type=jnp.float32) + f1b_ref[:]
    h = 0.5 * h * (1.0 + lax.erf(h * jnp.float32(0.7071067811865476)))
    out_ref[:] = jnp.dot(h, f2w_ref[:], preferred_element_type=jnp.float32) + f2b_ref[:]


def kernel(F_patches, alpha, Wv_w, Wv_b, f1_w, f1_b, f2_w, f2_b, step):
    b, n, d = F_patches.shape
    wv = lax.dynamic_index_in_dim(Wv_w, step, 0, keepdims=False)
    bv = lax.dynamic_index_in_dim(Wv_b, step, 0, keepdims=False)

    sc1 = _sc_sums(F_patches, alpha)

    wts, tki, tkw = pl.pallas_call(
        _topk_kernel,
        out_shape=[jax.ShapeDtypeStruct((b, 4, n), jnp.float32),
                   jax.ShapeDtypeStruct((b, _TOP_K), jnp.int32),
                   jax.ShapeDtypeStruct((b, _TOP_K), jnp.float32)],
    )(alpha)

    sc2 = _sc_topk(F_patches, tki, tkw)

    acc = pl.pallas_call(
        _stream_kernel,
        grid=(_B_TC, 1),
        in_specs=[
            pl.BlockSpec((b, 4, n), lambda i, j: (0, 0, 0)),
            pl.BlockSpec((1, n, d), lambda i, j: (i, j, 0)),
        ],
        out_specs=pl.BlockSpec((1, 4, d), lambda i, j: (i, 0, 0)),
        out_shape=jax.ShapeDtypeStruct((_B_TC, 4, d), jnp.float32),
    )(wts, F_patches)

    out = pl.pallas_call(
        _finish_kernel,
        out_shape=jax.ShapeDtypeStruct((b, d), jnp.float32),
    )(acc, sc1, sc2, wts, wv, bv, f1_w, f1_b, f2_w, f2_b)
    return out


# speedup vs baseline: 1.0608x; 1.0608x over previous
import jax
import jax.numpy as jnp
from jax import lax
from jax.experimental import pallas as pl

_TOP_K = 32
_NBLK = 4096


def _topk_kernel(alpha_ref, wts_ref):
    a = alpha_ref[:]
    b, n = a.shape
    iota = lax.broadcasted_iota(jnp.int32, (b, n), 1)
    neg = jnp.float32(-jnp.inf)

    def body(_, carry):
        a, wnum, ind, s = carry
        m = jnp.max(a, axis=1, keepdims=True)
        idx = jnp.min(jnp.where(a == m, iota, n), axis=1, keepdims=True)
        onehot = iota == idx
        wnum = wnum + jnp.where(onehot, m, 0.0)
        ind = ind + jnp.where(onehot, 1.0, 0.0)
        s = s + m
        a = jnp.where(onehot, neg, a)
        return a, wnum, ind, s

    zeros = jnp.zeros_like(a)
    s0 = jnp.zeros((b, 1), jnp.float32)
    _, wnum, ind, s = lax.fori_loop(0, _TOP_K, body, (a, zeros, zeros, s0))
    wts_ref[:, 0, :] = jnp.ones_like(a)
    wts_ref[:, 1, :] = alpha_ref[:]
    wts_ref[:, 2, :] = wnum / (s + 1e-8)
    wts_ref[:, 3, :] = ind


def _stream_kernel(wts_ref, f_ref, acc_ref):
    i = pl.program_id(0)
    nb = pl.program_id(1)
    w = wts_ref[i].astype(jnp.bfloat16)
    part = jnp.dot(w, f_ref[0].astype(jnp.bfloat16),
                   preferred_element_type=jnp.float32)

    @pl.when(nb == 0)
    def _():
        acc_ref[0] = part

    @pl.when(nb != 0)
    def _():
        acc_ref[0] += part


def _finish_kernel(acc_ref, wts_ref, wv_ref, bv_ref, f1w_ref, f1b_ref,
                   f2w_ref, f2b_ref, out_ref):
    acc = acc_ref[:]
    b = acc.shape[0]
    n = wts_ref.shape[2]
    p = jnp.dot(acc.reshape(b * 4, -1), wv_ref[:],
                preferred_element_type=jnp.float32).reshape(b, 4, -1)
    bv = bv_ref[:]
    sum_alpha = jnp.sum(wts_ref[:, 1, :], axis=1, keepdims=True)
    sum_w = jnp.sum(wts_ref[:, 2, :], axis=1, keepdims=True)
    s_all = p[:, 0, :] + jnp.float32(n) * bv
    broad = p[:, 1, :] + sum_alpha * bv
    focus = p[:, 2, :] + sum_w * bv
    s_top = p[:, 3, :] + jnp.float32(_TOP_K) * bv
    periph = (s_all - s_top) * jnp.float32(1.0 / (n - _TOP_K))
    concat = jnp.concatenate([focus, broad, periph], axis=-1)
    h = jnp.dot(concat, f1w_ref[:], preferred_element_type=jnp.float32) + f1b_ref[:]
    h = 0.5 * h * (1.0 + lax.erf(h * jnp.float32(0.7071067811865476)))
    out_ref[:] = jnp.dot(h, f2w_ref[:], preferred_element_type=jnp.float32) + f2b_ref[:]


def kernel(F_patches, alpha, Wv_w, Wv_b, f1_w, f1_b, f2_w, f2_b, step):
    b, n, d = F_patches.shape
    wv = lax.dynamic_index_in_dim(Wv_w, step, 0, keepdims=False)
    bv = lax.dynamic_index_in_dim(Wv_b, step, 0, keepdims=False)

    wts = pl.pallas_call(
        _topk_kernel,
        out_shape=jax.ShapeDtypeStruct((b, 4, n), jnp.float32),
    )(alpha)

    nb = n // _NBLK
    acc = pl.pallas_call(
        _stream_kernel,
        grid=(b, nb),
        in_specs=[
            pl.BlockSpec((b, 4, n), lambda i, j: (0, 0, 0)),
            pl.BlockSpec((1, _NBLK, d), lambda i, j: (i, j, 0)),
        ],
        out_specs=pl.BlockSpec((1, 4, d), lambda i, j: (i, 0, 0)),
        out_shape=jax.ShapeDtypeStruct((b, 4, d), jnp.float32),
    )(wts, F_patches)

    out = pl.pallas_call(
        _finish_kernel,
        out_shape=jax.ShapeDtypeStruct((b, d), jnp.float32),
    )(acc, wts, wv, bv, f1_w, f1_b, f2_w, f2_b)
    return out
